# baseline (speedup 1.0000x reference)
import functools

import jax
import jax.numpy as jnp
from jax.experimental import pallas as pl
from jax.experimental.pallas import tpu as pltpu


_SUBSPLIT = 4


def _ln(x, g, b, eps=1e-5):
    mu = jnp.mean(x, axis=-1, keepdims=True)
    var = jnp.mean((x - mu) ** 2, axis=-1, keepdims=True)
    return (x - mu) / jnp.sqrt(var + eps) * g + b


def _pick_tile(n):
    for t in (15000, 12000, 8000, 6000, 4800, 4000, 3000, 2400, 2000, 1600, 1200, 1000, 800,
              600, 480, 400, 240, 200, 160, 120, 96, 80, 64, 48, 40, 32,
              24, 16, 8):
        if n % t == 0:
            return t
    return n


def _fused_body(cc_ref, pts_ref, ln0g_ref, ln0b_ref, W1_ref, b1_ref,
                W2_ref, b2_ref, W3_ref, b3_ref, bln1g_ref, bln1b_ref,
                Wb_ref, bln2g_ref, bln2b_ref,
                logits_ref, out_ref, me_ref, acc_ref, *, tile, num_q):
    i = pl.program_id(0)

    @pl.when(i == 0)
    def _():
        h = _ln(cc_ref[...], ln0g_ref[...], ln0b_ref[...])
        h = jnp.maximum(
            jax.lax.dot_general(h, W1_ref[...], (((1,), (1,)), ((), ())),
                                preferred_element_type=jnp.float32)
            + b1_ref[...], 0.0)
        h = jnp.maximum(
            jax.lax.dot_general(h, W2_ref[...], (((1,), (1,)), ((), ())),
                                preferred_element_type=jnp.float32)
            + b2_ref[...], 0.0)
        me_ref[...] = jax.lax.dot_general(
            h, W3_ref[...], (((1,), (1,)), ((), ())),
            preferred_element_type=jnp.float32) + b3_ref[...]
        acc_ref[...] = jnp.zeros_like(acc_ref)

    sub = tile // _SUBSPLIT
    partials = []
    for s in range(_SUBSPLIT):
        pts = pts_ref[pl.ds(s * sub, sub), :]
        logits = jax.lax.dot_general(
            pts, me_ref[...], (((1,), (1,)), ((), ())),
            preferred_element_type=jnp.float32)
        logits_ref[pl.ds(s * sub, sub), :] = logits
        rowmax = jnp.max(logits, axis=1, keepdims=True)
        onehot = (logits == rowmax).astype(jnp.bfloat16)
        partials.append(jax.lax.dot_general(
            onehot, pts.astype(jnp.bfloat16), (((0,), (0,)), ((), ())),
            preferred_element_type=jnp.float32))
    acc_ref[...] += sum(partials)

    @pl.when(i == pl.num_programs(0) - 1)
    def _():
        m = _ln(acc_ref[...], bln1g_ref[...], bln1b_ref[...])
        m = jax.lax.dot_general(m, Wb_ref[...], (((1,), (1,)), ((), ())),
                                preferred_element_type=jnp.float32)
        m = _ln(m, bln2g_ref[...], bln2b_ref[...])
        out_ref[...] = cc_ref[...] + m


def kernel(cluster_centers, point_features, ln0_g, ln0_b, W1, b1, W2, b2,
           W3, b3, bln1_g, bln1_b, Wb, bln2_g, bln2_b):
    n, d = point_features.shape
    num_q = cluster_centers.shape[0]
    tile = _pick_tile(n)
    grid = n // tile

    row = lambda v: v.reshape(1, d).astype(jnp.float32)
    const_spec = lambda shape: pl.BlockSpec(shape, lambda i: (0, 0))

    body = functools.partial(_fused_body, tile=tile, num_q=num_q)
    pred_logits, new_centers = pl.pallas_call(
        body,
        grid=(grid,),
        in_specs=[
            const_spec((num_q, d)),
            pl.BlockSpec((tile, d), lambda i: (i, 0)),
            const_spec((1, d)), const_spec((1, d)),
            const_spec((d, d)), const_spec((1, d)),
            const_spec((d, d)), const_spec((1, d)),
            const_spec((d, d)), const_spec((1, d)),
            const_spec((1, d)), const_spec((1, d)),
            const_spec((d, d)),
            const_spec((1, d)), const_spec((1, d)),
        ],
        out_specs=[
            pl.BlockSpec((tile, num_q), lambda i: (i, 0)),
            const_spec((num_q, d)),
        ],
        out_shape=[
            jax.ShapeDtypeStruct((n, num_q), jnp.float32),
            jax.ShapeDtypeStruct((num_q, d), jnp.float32),
        ],
        scratch_shapes=[
            pltpu.VMEM((num_q, d), jnp.float32),
            pltpu.VMEM((num_q, d), jnp.float32),
        ],
        compiler_params=pltpu.CompilerParams(
            dimension_semantics=("arbitrary",)),
    )(cluster_centers, point_features, row(ln0_g), row(ln0_b),
      W1, row(b1), W2, row(b2), W3, row(b3),
      row(bln1_g), row(bln1_b), Wb, row(bln2_g), row(bln2_b))
    return (pred_logits, new_centers)


# speedup vs baseline: 1.0405x; 1.0405x over previous
import functools

import jax
import jax.numpy as jnp
from jax.experimental import pallas as pl
from jax.experimental.pallas import tpu as pltpu


_SUBSPLIT = 1


def _ln(x, g, b, eps=1e-5):
    mu = jnp.mean(x, axis=-1, keepdims=True)
    var = jnp.mean((x - mu) ** 2, axis=-1, keepdims=True)
    return (x - mu) / jnp.sqrt(var + eps) * g + b


def _pick_tile(n):
    for t in (15000, 12000, 8000, 6000, 4800, 4000, 3000, 2400, 2000, 1600, 1200, 1000, 800,
              600, 480, 400, 240, 200, 160, 120, 96, 80, 64, 48, 40, 32,
              24, 16, 8):
        if n % t == 0:
            return t
    return n


def _fused_body(cc_ref, pts_ref, ln0g_ref, ln0b_ref, W1_ref, b1_ref,
                W2_ref, b2_ref, W3_ref, b3_ref, bln1g_ref, bln1b_ref,
                Wb_ref, bln2g_ref, bln2b_ref,
                logits_ref, out_ref, me_ref, acc_ref, *, tile, num_q):
    i = pl.program_id(0)

    @pl.when(i == 0)
    def _():
        h = _ln(cc_ref[...], ln0g_ref[...], ln0b_ref[...])
        h = jnp.maximum(
            jax.lax.dot_general(h, W1_ref[...], (((1,), (1,)), ((), ())),
                                preferred_element_type=jnp.float32)
            + b1_ref[...], 0.0)
        h = jnp.maximum(
            jax.lax.dot_general(h, W2_ref[...], (((1,), (1,)), ((), ())),
                                preferred_element_type=jnp.float32)
            + b2_ref[...], 0.0)
        me_ref[...] = jax.lax.dot_general(
            h, W3_ref[...], (((1,), (1,)), ((), ())),
            preferred_element_type=jnp.float32) + b3_ref[...]
        acc_ref[...] = jnp.zeros_like(acc_ref)

    sub = tile // _SUBSPLIT
    partials = []
    for s in range(_SUBSPLIT):
        pts = pts_ref[pl.ds(s * sub, sub), :]
        logits = jax.lax.dot_general(
            pts, me_ref[...], (((1,), (1,)), ((), ())),
            preferred_element_type=jnp.float32)
        logits_ref[pl.ds(s * sub, sub), :] = logits
        rowmax = jnp.max(logits, axis=1, keepdims=True)
        onehot = (logits == rowmax).astype(jnp.bfloat16)
        partials.append(jax.lax.dot_general(
            onehot, pts.astype(jnp.bfloat16), (((0,), (0,)), ((), ())),
            preferred_element_type=jnp.float32))
    acc_ref[...] += sum(partials)

    @pl.when(i == pl.num_programs(0) - 1)
    def _():
        m = _ln(acc_ref[...], bln1g_ref[...], bln1b_ref[...])
        m = jax.lax.dot_general(m, Wb_ref[...], (((1,), (1,)), ((), ())),
                                preferred_element_type=jnp.float32)
        m = _ln(m, bln2g_ref[...], bln2b_ref[...])
        out_ref[...] = cc_ref[...] + m


def kernel(cluster_centers, point_features, ln0_g, ln0_b, W1, b1, W2, b2,
           W3, b3, bln1_g, bln1_b, Wb, bln2_g, bln2_b):
    n, d = point_features.shape
    num_q = cluster_centers.shape[0]
    tile = _pick_tile(n)
    grid = n // tile

    row = lambda v: v.reshape(1, d).astype(jnp.float32)
    const_spec = lambda shape: pl.BlockSpec(shape, lambda i: (0, 0))

    body = functools.partial(_fused_body, tile=tile, num_q=num_q)
    pred_logits, new_centers = pl.pallas_call(
        body,
        grid=(grid,),
        in_specs=[
            const_spec((num_q, d)),
            pl.BlockSpec((tile, d), lambda i: (i, 0)),
            const_spec((1, d)), const_spec((1, d)),
            const_spec((d, d)), const_spec((1, d)),
            const_spec((d, d)), const_spec((1, d)),
            const_spec((d, d)), const_spec((1, d)),
            const_spec((1, d)), const_spec((1, d)),
            const_spec((d, d)),
            const_spec((1, d)), const_spec((1, d)),
        ],
        out_specs=[
            pl.BlockSpec((tile, num_q), lambda i: (i, 0)),
            const_spec((num_q, d)),
        ],
        out_shape=[
            jax.ShapeDtypeStruct((n, num_q), jnp.float32),
            jax.ShapeDtypeStruct((num_q, d), jnp.float32),
        ],
        scratch_shapes=[
            pltpu.VMEM((num_q, d), jnp.float32),
            pltpu.VMEM((num_q, d), jnp.float32),
        ],
        compiler_params=pltpu.CompilerParams(
            dimension_semantics=("arbitrary",)),
    )(cluster_centers, point_features, row(ln0_g), row(ln0_b),
      W1, row(b1), W2, row(b2), W3, row(b3),
      row(bln1_g), row(bln1_b), Wb, row(bln2_g), row(bln2_b))
    return (pred_logits, new_centers)


# speedup vs baseline: 1.1656x; 1.1202x over previous
import functools

import jax
import jax.numpy as jnp
from jax.experimental import pallas as pl
from jax.experimental.pallas import tpu as pltpu


_SUBSPLIT = 2


def _ln(x, g, b, eps=1e-5):
    mu = jnp.mean(x, axis=-1, keepdims=True)
    var = jnp.mean((x - mu) ** 2, axis=-1, keepdims=True)
    return (x - mu) / jnp.sqrt(var + eps) * g + b


def _pick_tile(n):
    for t in (20000, 15000, 12000, 8000, 6000, 4800, 4000, 3000, 2400, 2000, 1600, 1200, 1000, 800,
              600, 480, 400, 240, 200, 160, 120, 96, 80, 64, 48, 40, 32,
              24, 16, 8):
        if n % t == 0:
            return t
    return n


def _fused_body(cc_ref, pts_ref, ln0g_ref, ln0b_ref, W1_ref, b1_ref,
                W2_ref, b2_ref, W3_ref, b3_ref, bln1g_ref, bln1b_ref,
                Wb_ref, bln2g_ref, bln2b_ref,
                logits_ref, out_ref, me_ref, acc_ref, *, tile, num_q):
    i = pl.program_id(0)

    @pl.when(i == 0)
    def _():
        h = _ln(cc_ref[...], ln0g_ref[...], ln0b_ref[...])
        h = jnp.maximum(
            jax.lax.dot_general(h, W1_ref[...], (((1,), (1,)), ((), ())),
                                preferred_element_type=jnp.float32)
            + b1_ref[...], 0.0)
        h = jnp.maximum(
            jax.lax.dot_general(h, W2_ref[...], (((1,), (1,)), ((), ())),
                                preferred_element_type=jnp.float32)
            + b2_ref[...], 0.0)
        me_ref[...] = jax.lax.dot_general(
            h, W3_ref[...], (((1,), (1,)), ((), ())),
            preferred_element_type=jnp.float32) + b3_ref[...]
        acc_ref[...] = jnp.zeros_like(acc_ref)

    sub = tile // _SUBSPLIT
    partials = []
    for s in range(_SUBSPLIT):
        pts = pts_ref[pl.ds(s * sub, sub), :]
        logits = jax.lax.dot_general(
            pts, me_ref[...], (((1,), (1,)), ((), ())),
            preferred_element_type=jnp.float32)
        logits_ref[pl.ds(s * sub, sub), :] = logits
        rowmax = jnp.max(logits, axis=1, keepdims=True)
        onehot = (logits == rowmax).astype(jnp.bfloat16)
        partials.append(jax.lax.dot_general(
            onehot, pts.astype(jnp.bfloat16), (((0,), (0,)), ((), ())),
            preferred_element_type=jnp.float32))
    acc_ref[...] += sum(partials)

    @pl.when(i == pl.num_programs(0) - 1)
    def _():
        m = _ln(acc_ref[...], bln1g_ref[...], bln1b_ref[...])
        m = jax.lax.dot_general(m, Wb_ref[...], (((1,), (1,)), ((), ())),
                                preferred_element_type=jnp.float32)
        m = _ln(m, bln2g_ref[...], bln2b_ref[...])
        out_ref[...] = cc_ref[...] + m


def kernel(cluster_centers, point_features, ln0_g, ln0_b, W1, b1, W2, b2,
           W3, b3, bln1_g, bln1_b, Wb, bln2_g, bln2_b):
    n, d = point_features.shape
    num_q = cluster_centers.shape[0]
    tile = _pick_tile(n)
    grid = n // tile

    row = lambda v: v.reshape(1, d).astype(jnp.float32)
    const_spec = lambda shape: pl.BlockSpec(shape, lambda i: (0, 0))

    body = functools.partial(_fused_body, tile=tile, num_q=num_q)
    pred_logits, new_centers = pl.pallas_call(
        body,
        grid=(grid,),
        in_specs=[
            const_spec((num_q, d)),
            pl.BlockSpec((tile, d), lambda i: (i, 0)),
            const_spec((1, d)), const_spec((1, d)),
            const_spec((d, d)), const_spec((1, d)),
            const_spec((d, d)), const_spec((1, d)),
            const_spec((d, d)), const_spec((1, d)),
            const_spec((1, d)), const_spec((1, d)),
            const_spec((d, d)),
            const_spec((1, d)), const_spec((1, d)),
        ],
        out_specs=[
            pl.BlockSpec((tile, num_q), lambda i: (i, 0)),
            const_spec((num_q, d)),
        ],
        out_shape=[
            jax.ShapeDtypeStruct((n, num_q), jnp.float32),
            jax.ShapeDtypeStruct((num_q, d), jnp.float32),
        ],
        scratch_shapes=[
            pltpu.VMEM((num_q, d), jnp.float32),
            pltpu.VMEM((num_q, d), jnp.float32),
        ],
        compiler_params=pltpu.CompilerParams(
            dimension_semantics=("arbitrary",)),
    )(cluster_centers, point_features, row(ln0_g), row(ln0_b),
      W1, row(b1), W2, row(b2), W3, row(b3),
      row(bln1_g), row(bln1_b), Wb, row(bln2_g), row(bln2_b))
    return (pred_logits, new_centers)
